# baseline (speedup 1.0000x reference)
import functools

import jax
import jax.numpy as jnp
from jax import lax
from jax.experimental import pallas as pl
from jax.experimental.pallas import tpu as pltpu
from jax.experimental.pallas import tpu_sc as plsc

D = 64
L = 16
NC = 2
NS = 16
NW = NC * NS
C = 224
S = 3


@jax.jit
def _run(x_user, x_movie, eli):
    E = eli.shape[1]
    eli = eli.astype(jnp.int32)
    Tf = E // (NW * C)
    T = Tf + 1
    T += (-T) % S
    n_tail = T - Tf
    assert n_tail * NW * C >= E - Tf * NW * C
    assert NW * C <= E and Tf >= S
    mesh = plsc.VectorSubcoreMesh(core_axis_name="c", subcore_axis_name="s")

    @functools.partial(
        pl.kernel,
        mesh=mesh,
        compiler_params=pltpu.CompilerParams(
            needs_layout_passes=False, use_tc_tiling_on_sc=False
        ),
        out_type=jax.ShapeDtypeStruct((E,), jnp.float32),
        scratch_types=[
            pltpu.VMEM((T * C,), jnp.int32),
            pltpu.VMEM((T * C,), jnp.int32),
        ]
        + [pltpu.VMEM((C, D), jnp.float32) for _ in range(2 * S)]
        + [pltpu.VMEM((C,), jnp.float32) for _ in range(S)]
        + [pltpu.SemaphoreType.DMA for _ in range(2 * S)],
    )
    def k(xu, xm, ei, out, iu_all, im_all, *bufs_flat):
        rows = bufs_flat[: 2 * S]
        outs = bufs_flat[2 * S: 3 * S]
        sems = bufs_flat[3 * S:]
        bufs = tuple(
            (rows[2 * b], rows[2 * b + 1], outs[b], sems[2 * b], sems[2 * b + 1])
            for b in range(S)
        )
        wid = lax.axis_index("s") * NC + lax.axis_index("c")

        def hbase(t):
            return jnp.where(
                t < Tf,
                (wid * Tf + t) * C,
                E - ((T - t) * NW - wid) * C,
            )

        pltpu.sync_copy(ei.at[0, pl.ds(wid * Tf * C, Tf * C)],
                        iu_all.at[pl.ds(0, Tf * C)])
        pltpu.sync_copy(ei.at[1, pl.ds(wid * Tf * C, Tf * C)],
                        im_all.at[pl.ds(0, Tf * C)])
        for j in range(n_tail):
            tb = E - ((n_tail - j) * NW - wid) * C
            pltpu.sync_copy(ei.at[0, pl.ds(tb, C)],
                            iu_all.at[pl.ds((Tf + j) * C, C)])
            pltpu.sync_copy(ei.at[1, pl.ds(tb, C)],
                            im_all.at[pl.ds((Tf + j) * C, C)])

        def gathers(t, b):
            ruv, rmv, _, sg, _ = bufs[b]
            cu = pltpu.make_async_copy(xu.at[iu_all.at[pl.ds(t * C, C)]], ruv, sg)
            cm = pltpu.make_async_copy(xm.at[im_all.at[pl.ds(t * C, C)]], rmv, sg)
            return cu, cm

        for b in range(S):
            cu, cm = gathers(b, b)
            cu.start()
            cm.start()

        lanes = lax.iota(jnp.int32, L)
        iE = (lanes % (L // 2)) * 2
        iO = iE + 1
        lo_mask = lanes < (L // 2)

        def _perm(a, idx_):
            return jnp.take_along_axis(a, idx_, axis=0)

        def _hadd(a, b):
            ta = _perm(a, iE) + _perm(a, iO)
            tb = _perm(b, iE) + _perm(b, iO)
            return jnp.where(lo_mask, ta, tb)

        def outer(i, carry):
            t0 = i * S
            for b in range(S):
                ruv, rmv, ov, sg, so = bufs[b]
                t = t0 + b
                cu, cm = gathers(t, b)
                cu.wait()
                cm.wait()

                @pl.when(t >= S)
                def _():
                    pltpu.make_async_copy(
                        ov, out.at[pl.ds(hbase(t - S), C)], so
                    ).wait()

                def group_body(g, carry2):
                    eb = g * L
                    p = []
                    for j in range(L):
                        e = eb + j
                        v = ruv[e, pl.ds(0, L)] * rmv[e, pl.ds(0, L)]
                        for q in range(1, D // L):
                            v = v + ruv[e, pl.ds(q * L, L)] * rmv[e, pl.ds(q * L, L)]
                        p.append(v)
                    while len(p) > 1:
                        p = [_hadd(p[i2], p[i2 + 1]) for i2 in range(0, len(p), 2)]
                    ov[pl.ds(eb, L)] = p[0]
                    return carry2

                lax.fori_loop(0, C // L, group_body, 0)
                pltpu.make_async_copy(ov, out.at[pl.ds(hbase(t), C)], so).start()

                @pl.when(t + S < T)
                def _():
                    cu2, cm2 = gathers(t + S, b)
                    cu2.start()
                    cm2.start()

            return carry

        lax.fori_loop(0, T // S, outer, 0)

        for b in range(S):
            _, _, ov, _, so = bufs[b]
            t = T - S + b
            pltpu.make_async_copy(ov, out.at[pl.ds(hbase(t), C)], so).wait()

    return k(x_user, x_movie, eli)


def kernel(x_user, x_movie, edge_label_index):
    return _run(x_user, x_movie, edge_label_index)


# speedup vs baseline: 1.0527x; 1.0527x over previous
import functools

import jax
import jax.numpy as jnp
from jax import lax
from jax.experimental import pallas as pl
from jax.experimental.pallas import tpu as pltpu
from jax.experimental.pallas import tpu_sc as plsc

D = 64
L = 16
NC = 2
NS = 16
NW = NC * NS
C = 320


@jax.jit
def _run(x_user, x_movie, eli):
    E = eli.shape[1]
    eli = eli.astype(jnp.int32)
    Tf = E // (NW * C)
    T = Tf + 1
    T += T % 2
    assert NW * C <= E
    mesh = plsc.VectorSubcoreMesh(core_axis_name="c", subcore_axis_name="s")

    @functools.partial(
        pl.kernel,
        mesh=mesh,
        compiler_params=pltpu.CompilerParams(
            needs_layout_passes=False, use_tc_tiling_on_sc=False
        ),
        out_type=jax.ShapeDtypeStruct((E,), jnp.float32),
        scratch_types=[
            pltpu.VMEM((T * C,), jnp.int32),
            pltpu.VMEM((T * C,), jnp.int32),
            pltpu.VMEM((C, D), jnp.float32),
            pltpu.VMEM((C, D), jnp.float32),
            pltpu.VMEM((C, D), jnp.float32),
            pltpu.VMEM((C, D), jnp.float32),
            pltpu.VMEM((C,), jnp.float32),
            pltpu.VMEM((C,), jnp.float32),
            pltpu.SemaphoreType.DMA,
            pltpu.SemaphoreType.DMA,
            pltpu.SemaphoreType.DMA,
            pltpu.SemaphoreType.DMA,
        ],
    )
    def k(xu, xm, ei, out, iu_all, im_all,
          ru0, ru1, rm0, rm1, o0, o1, sg0, sg1, so0, so1):
        wid = lax.axis_index("s") * NC + lax.axis_index("c")
        n_tail = T - Tf

        def hbase(t):
            return jnp.where(
                t < Tf,
                (wid * Tf + t) * C,
                E - ((T - t) * NW - wid) * C,
            )

        pltpu.sync_copy(ei.at[0, pl.ds(wid * Tf * C, Tf * C)],
                        iu_all.at[pl.ds(0, Tf * C)])
        pltpu.sync_copy(ei.at[1, pl.ds(wid * Tf * C, Tf * C)],
                        im_all.at[pl.ds(0, Tf * C)])
        for j in range(n_tail):
            tb = E - ((n_tail - j) * NW - wid) * C
            pltpu.sync_copy(ei.at[0, pl.ds(tb, C)],
                            iu_all.at[pl.ds((Tf + j) * C, C)])
            pltpu.sync_copy(ei.at[1, pl.ds(tb, C)],
                            im_all.at[pl.ds((Tf + j) * C, C)])

        bufs = ((ru0, rm0, o0, sg0, so0), (ru1, rm1, o1, sg1, so1))

        def gathers(t, b):
            ruv, rmv, _, sg, _ = bufs[b]
            cu = pltpu.make_async_copy(xu.at[iu_all.at[pl.ds(t * C, C)]], ruv, sg)
            cm = pltpu.make_async_copy(xm.at[im_all.at[pl.ds(t * C, C)]], rmv, sg)
            return cu, cm

        for b in range(2):
            cu, cm = gathers(b, b)
            cu.start()
            cm.start()

        lanes = lax.iota(jnp.int32, L)
        iE = (lanes % (L // 2)) * 2
        iO = iE + 1
        lo_mask = lanes < (L // 2)

        def _perm(a, idx_):
            return jnp.take_along_axis(a, idx_, axis=0)

        def _hadd(a, b):
            ta = _perm(a, iE) + _perm(a, iO)
            tb = _perm(b, iE) + _perm(b, iO)
            return jnp.where(lo_mask, ta, tb)

        def outer(i, carry):
            t0 = i * 2
            for b in range(2):
                ruv, rmv, ov, sg, so = bufs[b]
                t = t0 + b
                cu, cm = gathers(t, b)
                cu.wait()
                cm.wait()

                @pl.when(t >= 2)
                def _():
                    pltpu.make_async_copy(
                        ov, out.at[pl.ds(hbase(t - 2), C)], so
                    ).wait()

                def group_body(g, carry2):
                    eb = g * L
                    p = []
                    for j in range(L):
                        e = eb + j
                        v = ruv[e, pl.ds(0, L)] * rmv[e, pl.ds(0, L)]
                        for q in range(1, D // L):
                            v = v + ruv[e, pl.ds(q * L, L)] * rmv[e, pl.ds(q * L, L)]
                        p.append(v)
                    while len(p) > 1:
                        p = [_hadd(p[i2], p[i2 + 1]) for i2 in range(0, len(p), 2)]
                    ov[pl.ds(eb, L)] = p[0]
                    return carry2

                lax.fori_loop(0, C // L, group_body, 0)
                pltpu.make_async_copy(ov, out.at[pl.ds(hbase(t), C)], so).start()

                @pl.when(t + 2 < T)
                def _():
                    cu2, cm2 = gathers(t + 2, b)
                    cu2.start()
                    cm2.start()

            return carry

        lax.fori_loop(0, T // 2, outer, 0)

        for b in range(2):
            _, _, ov, _, so = bufs[b]
            t = T - 2 + b
            pltpu.make_async_copy(ov, out.at[pl.ds(hbase(t), C)], so).wait()

    return k(x_user, x_movie, eli)


def kernel(x_user, x_movie, edge_label_index):
    return _run(x_user, x_movie, edge_label_index)


# speedup vs baseline: 1.0538x; 1.0011x over previous
import functools

import jax
import jax.numpy as jnp
from jax import lax
from jax.experimental import pallas as pl
from jax.experimental.pallas import tpu as pltpu
from jax.experimental.pallas import tpu_sc as plsc

D = 64
L = 16
NC = 2
NS = 16
NW = NC * NS
C = 352


@jax.jit
def _run(x_user, x_movie, eli):
    E = eli.shape[1]
    eli = eli.astype(jnp.int32)
    Tf = E // (NW * C)
    T = Tf + 1
    T += T % 2
    assert NW * C <= E
    mesh = plsc.VectorSubcoreMesh(core_axis_name="c", subcore_axis_name="s")

    @functools.partial(
        pl.kernel,
        mesh=mesh,
        compiler_params=pltpu.CompilerParams(
            needs_layout_passes=False, use_tc_tiling_on_sc=False
        ),
        out_type=jax.ShapeDtypeStruct((E,), jnp.float32),
        scratch_types=[
            pltpu.VMEM((T * C,), jnp.int32),
            pltpu.VMEM((T * C,), jnp.int32),
            pltpu.VMEM((C, D), jnp.float32),
            pltpu.VMEM((C, D), jnp.float32),
            pltpu.VMEM((C, D), jnp.float32),
            pltpu.VMEM((C, D), jnp.float32),
            pltpu.VMEM((C,), jnp.float32),
            pltpu.VMEM((C,), jnp.float32),
            pltpu.SemaphoreType.DMA,
            pltpu.SemaphoreType.DMA,
            pltpu.SemaphoreType.DMA,
            pltpu.SemaphoreType.DMA,
        ],
    )
    def k(xu, xm, ei, out, iu_all, im_all,
          ru0, ru1, rm0, rm1, o0, o1, sg0, sg1, so0, so1):
        wid = lax.axis_index("s") * NC + lax.axis_index("c")
        n_tail = T - Tf

        def hbase(t):
            return jnp.where(
                t < Tf,
                (wid * Tf + t) * C,
                E - ((T - t) * NW - wid) * C,
            )

        pltpu.sync_copy(ei.at[0, pl.ds(wid * Tf * C, Tf * C)],
                        iu_all.at[pl.ds(0, Tf * C)])
        pltpu.sync_copy(ei.at[1, pl.ds(wid * Tf * C, Tf * C)],
                        im_all.at[pl.ds(0, Tf * C)])
        for j in range(n_tail):
            tb = E - ((n_tail - j) * NW - wid) * C
            pltpu.sync_copy(ei.at[0, pl.ds(tb, C)],
                            iu_all.at[pl.ds((Tf + j) * C, C)])
            pltpu.sync_copy(ei.at[1, pl.ds(tb, C)],
                            im_all.at[pl.ds((Tf + j) * C, C)])

        bufs = ((ru0, rm0, o0, sg0, so0), (ru1, rm1, o1, sg1, so1))

        def gathers(t, b):
            ruv, rmv, _, sg, _ = bufs[b]
            cu = pltpu.make_async_copy(xu.at[iu_all.at[pl.ds(t * C, C)]], ruv, sg)
            cm = pltpu.make_async_copy(xm.at[im_all.at[pl.ds(t * C, C)]], rmv, sg)
            return cu, cm

        for b in range(2):
            cu, cm = gathers(b, b)
            cu.start()
            cm.start()

        lanes = lax.iota(jnp.int32, L)
        iE = (lanes % (L // 2)) * 2
        iO = iE + 1
        lo_mask = lanes < (L // 2)

        def _perm(a, idx_):
            return jnp.take_along_axis(a, idx_, axis=0)

        def _hadd(a, b):
            ta = _perm(a, iE) + _perm(a, iO)
            tb = _perm(b, iE) + _perm(b, iO)
            return jnp.where(lo_mask, ta, tb)

        def outer(i, carry):
            t0 = i * 2
            for b in range(2):
                ruv, rmv, ov, sg, so = bufs[b]
                t = t0 + b
                cu, cm = gathers(t, b)
                cu.wait()
                cm.wait()

                @pl.when(t >= 2)
                def _():
                    pltpu.make_async_copy(
                        ov, out.at[pl.ds(hbase(t - 2), C)], so
                    ).wait()

                def group_body(g, carry2):
                    eb = g * L
                    p = []
                    for j in range(L):
                        e = eb + j
                        v = ruv[e, pl.ds(0, L)] * rmv[e, pl.ds(0, L)]
                        for q in range(1, D // L):
                            v = v + ruv[e, pl.ds(q * L, L)] * rmv[e, pl.ds(q * L, L)]
                        p.append(v)
                    while len(p) > 1:
                        p = [_hadd(p[i2], p[i2 + 1]) for i2 in range(0, len(p), 2)]
                    ov[pl.ds(eb, L)] = p[0]
                    return carry2

                lax.fori_loop(0, C // L, group_body, 0)
                pltpu.make_async_copy(ov, out.at[pl.ds(hbase(t), C)], so).start()

                @pl.when(t + 2 < T)
                def _():
                    cu2, cm2 = gathers(t + 2, b)
                    cu2.start()
                    cm2.start()

            return carry

        lax.fori_loop(0, T // 2, outer, 0)

        for b in range(2):
            _, _, ov, _, so = bufs[b]
            t = T - 2 + b
            pltpu.make_async_copy(ov, out.at[pl.ds(hbase(t), C)], so).wait()

    return k(x_user, x_movie, eli)


def kernel(x_user, x_movie, edge_label_index):
    return _run(x_user, x_movie, edge_label_index)
